# baseline (speedup 1.0000x reference)
import functools

import jax
import jax.numpy as jnp
from jax import lax
from jax.experimental import pallas as pl
from jax.experimental.pallas import tpu as pltpu
from jax.experimental.pallas import tpu_sc as plsc

BR = 256
NC, NS = 2, 16
CG = 8
EPS = 1e-6

f32 = jnp.float32
i32 = jnp.int32


def _norm_body(x_ref, xn_ref, sl_ref, hf_ref):
    x = x_ref[...]
    d = x.shape[1]
    cw = d // CG
    ss = jnp.sum(x * x, axis=1, keepdims=True)
    n = jnp.maximum(jnp.sqrt(ss), 1e-12)
    xn_ref[...] = x / n
    for k in range(CG):
        sl_ref[k] = x[:, k * cw:(k + 1) * cw]
    hf_ref[0] = x[:, :128]
    hf_ref[1] = x[:, 128:]


def _normalize(x2d):
    n, d = x2d.shape
    cw = d // CG
    return pl.pallas_call(
        _norm_body,
        grid=(n // BR,),
        in_specs=[pl.BlockSpec((BR, d), lambda ib: (ib, 0))],
        out_specs=[
            pl.BlockSpec((BR, d), lambda ib: (ib, 0)),
            pl.BlockSpec((CG, BR, cw), lambda ib: (0, ib, 0)),
            pl.BlockSpec((2, BR, 128), lambda ib: (0, ib, 0)),
        ],
        out_shape=[
            jax.ShapeDtypeStruct((n, d), f32),
            jax.ShapeDtypeStruct((CG, n, cw), f32),
            jax.ShapeDtypeStruct((2, n, 128), f32),
        ],
    )(x2d)


def _tri_body(xf_ref, su_ref):
    xf = xf_ref[0]
    t, d = xf.shape
    xs = jnp.concatenate([xf[1:], xf[:1]], axis=0)
    prod = xf * xs
    ones = jnp.ones((1, d), f32)
    su = lax.dot_general(ones, prod, (((1,), (1,)), ((), ())),
                         preferred_element_type=f32)
    col = lax.broadcasted_iota(i32, (1, t), 1)
    su_ref[0] = jnp.where(col < t - 1, su, 0.0)


def _tridiag(xn3):
    b, t, d = xn3.shape
    return pl.pallas_call(
        _tri_body,
        grid=(b,),
        in_specs=[pl.BlockSpec((1, t, d), lambda ib: (ib, 0, 0))],
        out_specs=pl.BlockSpec((1, 1, t), lambda ib: (ib, 0, 0)),
        out_shape=jax.ShapeDtypeStruct((b, 1, t), f32),
    )(xn3)


def _simtop_body(xr_ref, xf_ref,
                 v1_ref, v2_ref, i1_ref, i2_ref, s_ref):
    t = xf_ref.shape[1]
    xr = xr_ref[0]
    xf = xf_ref[0]
    s_ref[...] = lax.dot_general(xf, xr, (((1,), (1,)), ((), ())),
                                 preferred_element_type=f32)
    s = s_ref[...]
    r0 = pl.program_id(1) * BR
    j_i = lax.broadcasted_iota(i32, (t, BR), 0)
    i_i = lax.broadcasted_iota(i32, (t, BR), 1) + r0
    diag = j_i == i_i
    sm = jnp.where(diag, -3.0, s)
    m1 = jnp.max(sm, axis=0, keepdims=True)
    i1 = jnp.min(jnp.where(sm == m1, j_i, t), axis=0, keepdims=True)
    sm2 = jnp.where(j_i == i1, -3.0, sm)
    m2 = jnp.max(sm2, axis=0, keepdims=True)
    i2 = jnp.min(jnp.where(sm2 == m2, j_i, t), axis=0, keepdims=True)
    v1_ref[0, 0] = m1
    v2_ref[0, 0] = m2
    i1_ref[0, 0] = i1
    i2_ref[0, 0] = i2


def _simtop(xn, b, t, d):
    nb = t // BR
    outs = pl.pallas_call(
        _simtop_body,
        grid=(b, nb),
        in_specs=[
            pl.BlockSpec((1, BR, d), lambda ib, ir: (ib, ir, 0)),
            pl.BlockSpec((1, t, d), lambda ib, ir: (ib, 0, 0)),
        ],
        out_specs=[pl.BlockSpec((1, 1, 1, BR), lambda ib, ir: (ib, ir, 0, 0))
                   for _ in range(4)],
        out_shape=[jax.ShapeDtypeStruct((b, nb, 1, BR), f32) for _ in range(2)]
        + [jax.ShapeDtypeStruct((b, nb, 1, BR), i32) for _ in range(2)],
        scratch_shapes=[pltpu.VMEM((t, BR), f32)],
    )(xn, xn)
    return [o.reshape(b, t) for o in outs]


def _make_k3(b, t):
    mesh = plsc.VectorSubcoreMesh(core_axis_name="c", subcore_axis_name="s",
                                  num_cores=NC, num_subcores=NS)
    out_type = [jax.ShapeDtypeStruct((b, t), f32) for _ in range(5)]
    scratch = ([pltpu.VMEM((t,), i32) for _ in range(2)]
               + [pltpu.VMEM((t,), f32) for _ in range(9)])

    @functools.partial(pl.kernel, mesh=mesh, out_type=out_type,
                       scratch_types=scratch,
                       compiler_params=pltpu.CompilerParams(
                           needs_layout_passes=False))
    def k3(i1_h, i2_h, v1_h, v2_h, sl_h, su_h,
           rs_h, w1_h, w2_h, u1_h, u2_h,
           i1_v, i2_v, v1_v, v2_v, sl_v, su_v,
           rs_v, w1_v, w2_v, u1_v, u2_v):
        cid = lax.axis_index("c")
        sid = lax.axis_index("s")
        wid = cid * NS + sid

        @pl.when(wid < b)
        def _():
            bi = wid
            pltpu.sync_copy(i1_h.at[bi], i1_v)
            pltpu.sync_copy(i2_h.at[bi], i2_v)
            pltpu.sync_copy(v1_h.at[bi], v1_v)
            pltpu.sync_copy(v2_h.at[bi], v2_v)
            pltpu.sync_copy(sl_h.at[bi], sl_v)
            pltpu.sync_copy(su_h.at[bi], su_v)

            def body(ch, carry):
                base = ch * 16
                ds = pl.ds(base, 16)
                row = base + lax.iota(i32, 16)
                i1c = i1_v[ds]
                i2c = i2_v[ds]
                o1 = (i1c != row - 1) & (i1c != row + 1)
                o2 = (i2c != row - 1) & (i2c != row + 1)
                t11 = plsc.load_gather(i1_v, [i1c])
                t12 = plsc.load_gather(i2_v, [i1c])
                t21 = plsc.load_gather(i1_v, [i2c])
                t22 = plsc.load_gather(i2_v, [i2c])
                mut1 = o1 & ((t11 == row) | (t12 == row))
                mut2 = o2 & ((t21 == row) | (t22 == row))
                v1c = v1_v[ds]
                v2c = v2_v[ds]
                zero = jnp.zeros((16,), f32)
                w1 = jnp.where(o1, v1c, zero)
                w2 = jnp.where(o2, v2c, zero)
                u1 = jnp.where(o1 & (~mut1), v1c, zero)
                u2 = jnp.where(o2 & (~mut2), v2c, zero)
                rs_v[ds] = (2.0 + EPS) + sl_v[ds] + su_v[ds] + w1 + w2
                w1_v[ds] = w1
                w2_v[ds] = w2
                u1_v[ds] = u1
                u2_v[ds] = u2
                return carry

            lax.fori_loop(0, t // 16, body, 0)

            lane = lax.iota(i32, 16)

            def body2(ch, carry):
                ds = pl.ds(ch * 16, 16)
                i1c = i1_v[ds]
                i2c = i2_v[ds]
                u1c = u1_v[ds]
                u2c = u2_v[ds]
                for l in range(16):
                    m = lane == l
                    plsc.addupdate_scatter(rs_v, [i1c], u1c, mask=m)
                    plsc.addupdate_scatter(rs_v, [i2c], u2c, mask=m)
                return carry

            lax.fori_loop(0, t // 16, body2, 0)

            pltpu.sync_copy(rs_v, rs_h.at[bi])
            pltpu.sync_copy(w1_v, w1_h.at[bi])
            pltpu.sync_copy(w2_v, w2_h.at[bi])
            pltpu.sync_copy(u1_v, u1_h.at[bi])
            pltpu.sync_copy(u2_v, u2_h.at[bi])

    return k3


def _rsqrt_body(rs_ref, q_ref):
    q_ref[...] = 1.0 / jnp.sqrt(rs_ref[...])


def _rsqrt(rs3):
    b, _, t = rs3.shape
    return pl.pallas_call(
        _rsqrt_body,
        grid=(b,),
        in_specs=[pl.BlockSpec((1, 1, t), lambda ib: (ib, 0, 0))],
        out_specs=pl.BlockSpec((1, 1, t), lambda ib: (ib, 0, 0)),
        out_shape=jax.ShapeDtypeStruct((b, 1, t), f32),
    )(rs3)


def _make_k5(b, t, d):
    cw = d // CG
    rc = 64
    ntask = b * CG
    nw = NC * NS
    tpw = ntask // nw
    mesh = plsc.VectorSubcoreMesh(core_axis_name="c", subcore_axis_name="s",
                                  num_cores=NC, num_subcores=NS)
    scratch = [
        pltpu.VMEM((t * cw,), f32),
        pltpu.VMEM((rc * cw,), f32),
        pltpu.VMEM((rc, 128), f32),
        pltpu.VMEM((rc, 128), f32),
        pltpu.VMEM((rc, 128), f32),
        pltpu.VMEM((rc, 128), f32),
        pltpu.VMEM((t,), f32),
    ] + [pltpu.VMEM((t,), f32) for _ in range(6)] \
      + [pltpu.VMEM((t,), i32) for _ in range(2)] \
      + [pltpu.VMEM((rc,), i32) for _ in range(4)] \
      + [pltpu.VMEM((rc,), f32) for _ in range(10)] \
      + [pltpu.SemaphoreType.DMA, pltpu.SemaphoreType.DMA]

    @functools.partial(pl.kernel, mesh=mesh,
                       out_type=jax.ShapeDtypeStruct((CG * b * t * cw,), f32),
                       scratch_types=scratch,
                       compiler_params=pltpu.CompilerParams(
                           needs_layout_passes=False))
    def k5(ysl_h, y128_h, q_h, i1_h, i2_h, sl_h, su_h, w1_h, w2_h,
           u1_h, u2_h,
           out_h,
           acc, y_v, g1a, g2a, g1b, g2b, qf_v,
           sl_v, su_v, w1_v, w2_v, u1_v, u2_v,
           i1_v, i2_v,
           x1a, x2a, x1b, x2b,
           cla, a1a, a2a, clb_, a1b_, a2b_, pl_v, pu_v, s1_v, s2_v,
           sema, semb):
        cid = lax.axis_index("c")
        sid = lax.axis_index("s")
        wid = sid * NC + cid

        def task(tk, carry):
            gid = wid * tpw + tk
            bi = gid // CG
            cg = gid - bi * CG
            fb = cg * (b * t) + bi * t
            half = cg // 4
            colbase = (cg - half * 4) * cw
            gb = half * (b * t) + bi * t

            pltpu.sync_copy(q_h.at[bi], qf_v)
            pltpu.sync_copy(i1_h.at[pl.ds(bi * t, t)], i1_v)
            pltpu.sync_copy(i2_h.at[pl.ds(bi * t, t)], i2_v)
            pltpu.sync_copy(sl_h.at[pl.ds(bi * t, t)], sl_v)
            pltpu.sync_copy(su_h.at[pl.ds(bi * t, t)], su_v)
            pltpu.sync_copy(w1_h.at[pl.ds(bi * t, t)], w1_v)
            pltpu.sync_copy(w2_h.at[pl.ds(bi * t, t)], w2_v)
            pltpu.sync_copy(u1_h.at[pl.ds(bi * t, t)], u1_v)
            pltpu.sync_copy(u2_h.at[pl.ds(bi * t, t)], u2_v)

            zero = jnp.zeros((16,), f32)

            def zr(z, carry2):
                acc[pl.ds(z * 16, 16)] = zero
                return carry2

            lax.fori_loop(0, t * cw // 16, zr, 0)

            nck = t // rc
            bufs = ((x1a, x2a, g1a, g2a, cla, a1a, a2a, sema),
                    (x1b, x2b, g1b, g2b, clb_, a1b_, a2b_, semb))

            def prep1(ck, bu):
                x1r, x2r, _, _, clr, a1r, a2r, _ = bu
                r0c = ck * rc

                def sp(sc, carry3):
                    dsl = pl.ds(sc * 16, 16)
                    dsg = pl.ds(r0c + sc * 16, 16)
                    qo = qf_v[dsg]
                    j1 = i1_v[dsg]
                    j2 = i2_v[dsg]
                    x1r[dsl] = j1 + gb
                    x2r[dsl] = j2 + gb
                    a1r[dsl] = w1_v[dsg] * plsc.load_gather(qf_v, [j1])
                    a2r[dsl] = w2_v[dsg] * plsc.load_gather(qf_v, [j2])
                    clr[dsl] = 2.0 * qo
                    return carry3

                lax.fori_loop(0, rc // 16, sp, 0)

            def issue1(bu):
                x1r, x2r, g1r, g2r, _, _, _, sm = bu
                pltpu.async_copy(y128_h.at[x1r], g1r, sm)
                pltpu.async_copy(y128_h.at[x2r], g2r, sm)

            def wait1(bu):
                x1r, x2r, g1r, g2r, _, _, _, sm = bu
                pltpu.make_async_copy(y128_h.at[x1r], g1r, sm).wait()
                pltpu.make_async_copy(y128_h.at[x2r], g2r, sm).wait()

            def rows1(ck, bu):
                _, _, g1r, g2r, clr, a1r, a2r, _ = bu
                r0c = ck * rc

                def row1(jl):
                    bc = jnp.full((16,), 0, i32) + jl
                    clv = plsc.load_gather(clr, [bc])
                    a1v = plsc.load_gather(a1r, [bc])
                    a2v = plsc.load_gather(a2r, [bc])
                    jbo = (r0c + jl) * cw
                    ylo = jl * cw
                    for v in range(cw // 16):
                        gsv = pl.ds(colbase + v * 16, 16)
                        plsc.addupdate(
                            acc.at[pl.ds(jbo + v * 16, 16)],
                            clv * y_v[pl.ds(ylo + v * 16, 16)]
                            + a1v * g1r[jl, gsv]
                            + a2v * g2r[jl, gsv])

                plsc.parallel_loop(0, rc, unroll=2)(row1)

            prep1(0, bufs[0])
            issue1(bufs[0])

            def prep2(ck):
                r0c = ck * rc

                def sp2(sc, carry3):
                    dsl = pl.ds(sc * 16, 16)
                    base = r0c + sc * 16
                    dsg = pl.ds(base, 16)
                    rloc = base + lax.iota(i32, 16)
                    qo = qf_v[dsg]
                    up = jnp.clip(rloc + 1, 0, t - 1)
                    dn = jnp.clip(rloc - 1, 0, t - 1)
                    pu_v[dsl] = jnp.where(rloc < t - 1,
                                          plsc.load_gather(sl_v, [up]),
                                          0.0) * qo
                    pl_v[dsl] = jnp.where(rloc > 0,
                                          plsc.load_gather(su_v, [dn]),
                                          0.0) * qo
                    s1_v[dsl] = u1_v[dsg] * qo
                    s2_v[dsl] = u2_v[dsg] * qo
                    return carry3

                lax.fori_loop(0, rc // 16, sp2, 0)

            def rows2(ck):
                r0c = ck * rc

                def r2(sc, carry3):
                    b16 = sc * 16
                    dsl = pl.ds(b16, 16)
                    pu16 = pu_v[dsl]
                    pl16 = pl_v[dsl]
                    s116 = s1_v[dsl]
                    s216 = s2_v[dsl]
                    j116 = i1_v[pl.ds(r0c + b16, 16)]
                    j216 = i2_v[pl.ds(r0c + b16, 16)]
                    for l in range(16):
                        jl = b16 + l
                        jb = r0c + jl
                        ylo = jl * cw
                        ju = jnp.minimum(jb + 1, t - 1)
                        jd = jnp.maximum(jb - 1, 0)
                        for v in range(cw // 16):
                            dsy = pl.ds(ylo + v * 16, 16)
                            yv = y_v[dsy]
                            plsc.addupdate(acc.at[pl.ds(ju * cw + v * 16, 16)],
                                           pu16[l] * yv)
                            plsc.addupdate(acc.at[pl.ds(jd * cw + v * 16, 16)],
                                           pl16[l] * yv)
                        for dst, cf in ((j116, s116), (j216, s216)):
                            cc = cf[l]

                            @pl.when(cc != 0.0)
                            def _():
                                ddo = dst[l] * cw
                                for v in range(cw // 16):
                                    dsa = pl.ds(ddo + v * 16, 16)
                                    dsy = pl.ds(ylo + v * 16, 16)
                                    plsc.addupdate(acc.at[dsa],
                                                   cc * y_v[dsy])
                    return carry3

                lax.fori_loop(0, rc // 16, r2, 0)

            def pair1(ckp, carry2):
                for p in range(2):
                    ck = ckp * 2 + p
                    nb = bufs[1 - p]

                    @pl.when(ck + 1 < nck)
                    def _():
                        prep1(ck + 1, nb)
                        issue1(nb)

                    pltpu.sync_copy(
                        ysl_h.at[pl.ds((fb + ck * rc) * cw, rc * cw)], y_v)
                    prep2(ck)
                    wait1(bufs[p])
                    rows1(ck, bufs[p])
                    rows2(ck)
                return carry2

            lax.fori_loop(0, nck // 2, pair1, 0)

            def rd(il):
                cf = plsc.load_gather(qf_v, [jnp.full((16,), 0, i32) + il])
                ilo = il * cw
                for v in range(cw // 16):
                    dsv = pl.ds(ilo + v * 16, 16)
                    acc[dsv] = cf * acc[dsv]

            plsc.parallel_loop(0, t, unroll=2)(rd)
            pltpu.sync_copy(acc, out_h.at[pl.ds(fb * cw, t * cw)])
            return carry

        lax.fori_loop(0, tpw, task, 0)

    return k5


def _make_dense_body(cg, cw, sliced_out):
    def body(a_ref, w_ref, b_ref, g_ref, be_ref, *outs):
        h = jnp.concatenate([a_ref[k] for k in range(cg)], axis=1)
        z = lax.dot_general(h, w_ref[...], (((1,), (1,)), ((), ())),
                            preferred_element_type=f32) + b_ref[...]
        z = jnp.where(z > 0, z, jnp.exp(z) - 1.0)
        mu = jnp.mean(z, axis=1, keepdims=True)
        zc = z - mu
        var = jnp.mean(zc * zc, axis=1, keepdims=True)
        o = zc / jnp.sqrt(var + 1e-5) * g_ref[...] + be_ref[...]
        if sliced_out:
            sl_ref, hf_ref = outs
            for k in range(cg):
                sl_ref[k] = o[:, k * cw:(k + 1) * cw]
            hf_ref[0] = o[:, :128]
            hf_ref[1] = o[:, 128:]
        else:
            outs[0][...] = o
    return body


def _dense(agg_sl, w, bb, g, be, sliced_out):
    cg, n, cw = agg_sl.shape
    d = cg * cw
    if sliced_out:
        out_specs = [
            pl.BlockSpec((CG, BR, cw), lambda ib: (0, ib, 0)),
            pl.BlockSpec((2, BR, 128), lambda ib: (0, ib, 0)),
        ]
        out_shape = [
            jax.ShapeDtypeStruct((CG, n, cw), f32),
            jax.ShapeDtypeStruct((2, n, 128), f32),
        ]
    else:
        out_specs = [pl.BlockSpec((BR, d), lambda ib: (ib, 0))]
        out_shape = [jax.ShapeDtypeStruct((n, d), f32)]
    outs = pl.pallas_call(
        _make_dense_body(cg, cw, sliced_out),
        grid=(n // BR,),
        in_specs=[
            pl.BlockSpec((CG, BR, cw), lambda ib: (0, ib, 0)),
            pl.BlockSpec((d, d), lambda ib: (0, 0)),
            pl.BlockSpec((1, d), lambda ib: (0, 0)),
            pl.BlockSpec((1, d), lambda ib: (0, 0)),
            pl.BlockSpec((1, d), lambda ib: (0, 0)),
        ],
        out_specs=out_specs,
        out_shape=out_shape,
    )(agg_sl, w, bb.reshape(1, d), g.reshape(1, d), be.reshape(1, d))
    return outs


def kernel(x, mask, W0, b0, g0, be0, W1, b1, g1, be1):
    b, t, d = x.shape
    bt = b * t
    cw = d // CG
    x2d = x.reshape(bt, d)

    xn, x_sl, x_half = _normalize(x2d)
    xn3 = xn.reshape(b, t, d)
    su = _tridiag(xn3).reshape(b, t)
    sl = jnp.concatenate([jnp.zeros((b, 1), f32), su[:, :-1]], axis=1)
    v1, v2, i1, i2 = _simtop(xn3, b, t, d)

    k3 = _make_k3(b, t)
    rs, w1, w2, u1, u2 = k3(i1, i2, v1, v2, sl, su)

    q = _rsqrt(rs.reshape(b, 1, t)).reshape(b, t)

    k5 = _make_k5(b, t, d)
    i1f = i1.reshape(bt)
    i2f = i2.reshape(bt)
    slf = sl.reshape(bt)
    suf = su.reshape(bt)
    w1f = w1.reshape(bt)
    w2f = w2.reshape(bt)
    u1f = u1.reshape(bt)
    u2f = u2.reshape(bt)

    def _agg(y_sl, y_half):
        return k5(y_sl.reshape(CG * bt * cw), y_half.reshape(2 * bt, 128),
                  q, i1f, i2f, slf, suf, w1f, w2f, u1f, u2f)

    agg1 = _agg(x_sl, x_half).reshape(CG, bt, cw)
    h1_sl, h1_half = _dense(agg1, W0, b0, g0, be0, sliced_out=True)
    agg2 = _agg(h1_sl, h1_half).reshape(CG, bt, cw)
    (h2,) = _dense(agg2, W1, b1, g1, be1, sliced_out=False)
    return h2.reshape(b, t, d)


# speedup vs baseline: 1.0260x; 1.0260x over previous
import functools

import jax
import jax.numpy as jnp
from jax import lax
from jax.experimental import pallas as pl
from jax.experimental.pallas import tpu as pltpu
from jax.experimental.pallas import tpu_sc as plsc

BR = 256
NC, NS = 2, 16
CG = 8
EPS = 1e-6

f32 = jnp.float32
i32 = jnp.int32


def _norm_body(x_ref, xn_ref, sl_ref, hf_ref):
    x = x_ref[...]
    d = x.shape[1]
    cw = d // CG
    ss = jnp.sum(x * x, axis=1, keepdims=True)
    n = jnp.maximum(jnp.sqrt(ss), 1e-12)
    xn_ref[...] = x / n
    for k in range(CG):
        sl_ref[k] = x[:, k * cw:(k + 1) * cw]
    hf_ref[0] = x[:, :128]
    hf_ref[1] = x[:, 128:]


def _normalize(x2d):
    n, d = x2d.shape
    cw = d // CG
    return pl.pallas_call(
        _norm_body,
        grid=(n // BR,),
        in_specs=[pl.BlockSpec((BR, d), lambda ib: (ib, 0))],
        out_specs=[
            pl.BlockSpec((BR, d), lambda ib: (ib, 0)),
            pl.BlockSpec((CG, BR, cw), lambda ib: (0, ib, 0)),
            pl.BlockSpec((2, BR, 128), lambda ib: (0, ib, 0)),
        ],
        out_shape=[
            jax.ShapeDtypeStruct((n, d), f32),
            jax.ShapeDtypeStruct((CG, n, cw), f32),
            jax.ShapeDtypeStruct((2, n, 128), f32),
        ],
    )(x2d)


def _tri_body(xf_ref, su_ref):
    xf = xf_ref[0]
    t, d = xf.shape
    xs = jnp.concatenate([xf[1:], xf[:1]], axis=0)
    prod = xf * xs
    ones = jnp.ones((1, d), f32)
    su = lax.dot_general(ones, prod, (((1,), (1,)), ((), ())),
                         preferred_element_type=f32)
    col = lax.broadcasted_iota(i32, (1, t), 1)
    su_ref[0] = jnp.where(col < t - 1, su, 0.0)


def _tridiag(xn3):
    b, t, d = xn3.shape
    return pl.pallas_call(
        _tri_body,
        grid=(b,),
        in_specs=[pl.BlockSpec((1, t, d), lambda ib: (ib, 0, 0))],
        out_specs=pl.BlockSpec((1, 1, t), lambda ib: (ib, 0, 0)),
        out_shape=jax.ShapeDtypeStruct((b, 1, t), f32),
    )(xn3)


def _simtop_body(xr_ref, xf_ref,
                 v1_ref, v2_ref, i1_ref, i2_ref, s_ref):
    t = xf_ref.shape[1]
    xr = xr_ref[0]
    xf = xf_ref[0]
    s_ref[...] = lax.dot_general(xf, xr, (((1,), (1,)), ((), ())),
                                 preferred_element_type=f32)
    s = s_ref[...]
    r0 = pl.program_id(1) * BR
    j_i = lax.broadcasted_iota(i32, (t, BR), 0)
    i_i = lax.broadcasted_iota(i32, (t, BR), 1) + r0
    diag = j_i == i_i
    sm = jnp.where(diag, -3.0, s)
    m1 = jnp.max(sm, axis=0, keepdims=True)
    i1 = jnp.min(jnp.where(sm == m1, j_i, t), axis=0, keepdims=True)
    sm2 = jnp.where(j_i == i1, -3.0, sm)
    m2 = jnp.max(sm2, axis=0, keepdims=True)
    i2 = jnp.min(jnp.where(sm2 == m2, j_i, t), axis=0, keepdims=True)
    v1_ref[0, 0] = m1
    v2_ref[0, 0] = m2
    i1_ref[0, 0] = i1
    i2_ref[0, 0] = i2


def _simtop(xn, b, t, d):
    nb = t // BR
    outs = pl.pallas_call(
        _simtop_body,
        grid=(b, nb),
        in_specs=[
            pl.BlockSpec((1, BR, d), lambda ib, ir: (ib, ir, 0)),
            pl.BlockSpec((1, t, d), lambda ib, ir: (ib, 0, 0)),
        ],
        out_specs=[pl.BlockSpec((1, 1, 1, BR), lambda ib, ir: (ib, ir, 0, 0))
                   for _ in range(4)],
        out_shape=[jax.ShapeDtypeStruct((b, nb, 1, BR), f32) for _ in range(2)]
        + [jax.ShapeDtypeStruct((b, nb, 1, BR), i32) for _ in range(2)],
        scratch_shapes=[pltpu.VMEM((t, BR), f32)],
    )(xn, xn)
    return [o.reshape(b, t) for o in outs]


def _make_k3(b, t):
    mesh = plsc.VectorSubcoreMesh(core_axis_name="c", subcore_axis_name="s",
                                  num_cores=NC, num_subcores=NS)
    out_type = [jax.ShapeDtypeStruct((b, t), f32) for _ in range(5)]
    scratch = ([pltpu.VMEM((t,), i32) for _ in range(2)]
               + [pltpu.VMEM((t,), f32) for _ in range(9)])

    @functools.partial(pl.kernel, mesh=mesh, out_type=out_type,
                       scratch_types=scratch,
                       compiler_params=pltpu.CompilerParams(
                           needs_layout_passes=False))
    def k3(i1_h, i2_h, v1_h, v2_h, sl_h, su_h,
           rs_h, w1_h, w2_h, u1_h, u2_h,
           i1_v, i2_v, v1_v, v2_v, sl_v, su_v,
           rs_v, w1_v, w2_v, u1_v, u2_v):
        cid = lax.axis_index("c")
        sid = lax.axis_index("s")
        wid = cid * NS + sid

        @pl.when(wid < b)
        def _():
            bi = wid
            pltpu.sync_copy(i1_h.at[bi], i1_v)
            pltpu.sync_copy(i2_h.at[bi], i2_v)
            pltpu.sync_copy(v1_h.at[bi], v1_v)
            pltpu.sync_copy(v2_h.at[bi], v2_v)
            pltpu.sync_copy(sl_h.at[bi], sl_v)
            pltpu.sync_copy(su_h.at[bi], su_v)

            def body(ch, carry):
                base = ch * 16
                ds = pl.ds(base, 16)
                row = base + lax.iota(i32, 16)
                i1c = i1_v[ds]
                i2c = i2_v[ds]
                o1 = (i1c != row - 1) & (i1c != row + 1)
                o2 = (i2c != row - 1) & (i2c != row + 1)
                t11 = plsc.load_gather(i1_v, [i1c])
                t12 = plsc.load_gather(i2_v, [i1c])
                t21 = plsc.load_gather(i1_v, [i2c])
                t22 = plsc.load_gather(i2_v, [i2c])
                mut1 = o1 & ((t11 == row) | (t12 == row))
                mut2 = o2 & ((t21 == row) | (t22 == row))
                v1c = v1_v[ds]
                v2c = v2_v[ds]
                zero = jnp.zeros((16,), f32)
                w1 = jnp.where(o1, v1c, zero)
                w2 = jnp.where(o2, v2c, zero)
                u1 = jnp.where(o1 & (~mut1), v1c, zero)
                u2 = jnp.where(o2 & (~mut2), v2c, zero)
                rs_v[ds] = (2.0 + EPS) + sl_v[ds] + su_v[ds] + w1 + w2
                w1_v[ds] = w1
                w2_v[ds] = w2
                u1_v[ds] = u1
                u2_v[ds] = u2
                return carry

            lax.fori_loop(0, t // 16, body, 0)

            lane = lax.iota(i32, 16)

            def body2(ch, carry):
                ds = pl.ds(ch * 16, 16)
                i1c = i1_v[ds]
                i2c = i2_v[ds]
                u1c = u1_v[ds]
                u2c = u2_v[ds]
                for l in range(16):
                    m = lane == l
                    plsc.addupdate_scatter(rs_v, [i1c], u1c, mask=m)
                    plsc.addupdate_scatter(rs_v, [i2c], u2c, mask=m)
                return carry

            lax.fori_loop(0, t // 16, body2, 0)

            pltpu.sync_copy(rs_v, rs_h.at[bi])
            pltpu.sync_copy(w1_v, w1_h.at[bi])
            pltpu.sync_copy(w2_v, w2_h.at[bi])
            pltpu.sync_copy(u1_v, u1_h.at[bi])
            pltpu.sync_copy(u2_v, u2_h.at[bi])

    return k3


def _rsqrt_body(rs_ref, q_ref):
    q_ref[...] = 1.0 / jnp.sqrt(rs_ref[...])


def _rsqrt(rs3):
    b, _, t = rs3.shape
    return pl.pallas_call(
        _rsqrt_body,
        grid=(b,),
        in_specs=[pl.BlockSpec((1, 1, t), lambda ib: (ib, 0, 0))],
        out_specs=pl.BlockSpec((1, 1, t), lambda ib: (ib, 0, 0)),
        out_shape=jax.ShapeDtypeStruct((b, 1, t), f32),
    )(rs3)


def _make_k5(b, t, d):
    cw = d // CG
    rc = 64
    ntask = b * CG
    nw = NC * NS
    tpw = ntask // nw
    mesh = plsc.VectorSubcoreMesh(core_axis_name="c", subcore_axis_name="s",
                                  num_cores=NC, num_subcores=NS)
    scratch = [
        pltpu.VMEM((t * cw,), f32),
        pltpu.VMEM((rc * cw,), f32),
        pltpu.VMEM((rc, 128), f32),
        pltpu.VMEM((rc, 128), f32),
        pltpu.VMEM((rc, 128), f32),
        pltpu.VMEM((rc, 128), f32),
        pltpu.VMEM((t,), f32),
    ] + [pltpu.VMEM((t,), f32) for _ in range(6)] \
      + [pltpu.VMEM((t,), i32) for _ in range(2)] \
      + [pltpu.VMEM((rc,), i32) for _ in range(4)] \
      + [pltpu.VMEM((rc,), f32) for _ in range(10)] \
      + [pltpu.SemaphoreType.DMA, pltpu.SemaphoreType.DMA]

    @functools.partial(pl.kernel, mesh=mesh,
                       out_type=jax.ShapeDtypeStruct((CG * b * t * cw,), f32),
                       scratch_types=scratch,
                       compiler_params=pltpu.CompilerParams(
                           needs_layout_passes=False))
    def k5(ysl_h, y128_h, q_h, i1_h, i2_h, sl_h, su_h, w1_h, w2_h,
           u1_h, u2_h,
           out_h,
           acc, y_v, g1a, g2a, g1b, g2b, qf_v,
           sl_v, su_v, w1_v, w2_v, u1_v, u2_v,
           i1_v, i2_v,
           x1a, x2a, x1b, x2b,
           cla, a1a, a2a, clb_, a1b_, a2b_, pl_v, pu_v, s1_v, s2_v,
           sema, semb):
        cid = lax.axis_index("c")
        sid = lax.axis_index("s")
        wid = sid * NC + cid

        def task(tk, carry):
            gid = wid * tpw + tk
            bi = gid // CG
            cg = gid - bi * CG
            fb = cg * (b * t) + bi * t
            half = cg // 4
            colbase = (cg - half * 4) * cw
            gb = half * (b * t) + bi * t

            pltpu.sync_copy(q_h.at[bi], qf_v)
            pltpu.sync_copy(i1_h.at[pl.ds(bi * t, t)], i1_v)
            pltpu.sync_copy(i2_h.at[pl.ds(bi * t, t)], i2_v)
            pltpu.sync_copy(sl_h.at[pl.ds(bi * t, t)], sl_v)
            pltpu.sync_copy(su_h.at[pl.ds(bi * t, t)], su_v)
            pltpu.sync_copy(w1_h.at[pl.ds(bi * t, t)], w1_v)
            pltpu.sync_copy(w2_h.at[pl.ds(bi * t, t)], w2_v)
            pltpu.sync_copy(u1_h.at[pl.ds(bi * t, t)], u1_v)
            pltpu.sync_copy(u2_h.at[pl.ds(bi * t, t)], u2_v)

            nck = t // rc
            bufs = ((x1a, x2a, g1a, g2a, cla, a1a, a2a, sema),
                    (x1b, x2b, g1b, g2b, clb_, a1b_, a2b_, semb))

            def prep1(ck, bu):
                x1r, x2r, _, _, clr, a1r, a2r, _ = bu
                r0c = ck * rc

                def sp(sc, carry3):
                    dsl = pl.ds(sc * 16, 16)
                    dsg = pl.ds(r0c + sc * 16, 16)
                    qo = qf_v[dsg]
                    j1 = i1_v[dsg]
                    j2 = i2_v[dsg]
                    x1r[dsl] = j1 + gb
                    x2r[dsl] = j2 + gb
                    a1r[dsl] = w1_v[dsg] * plsc.load_gather(qf_v, [j1])
                    a2r[dsl] = w2_v[dsg] * plsc.load_gather(qf_v, [j2])
                    clr[dsl] = 2.0 * qo
                    return carry3

                lax.fori_loop(0, rc // 16, sp, 0)

            def issue1(bu):
                x1r, x2r, g1r, g2r, _, _, _, sm = bu
                pltpu.async_copy(y128_h.at[x1r], g1r, sm)
                pltpu.async_copy(y128_h.at[x2r], g2r, sm)

            def wait1(bu):
                x1r, x2r, g1r, g2r, _, _, _, sm = bu
                pltpu.make_async_copy(y128_h.at[x1r], g1r, sm).wait()
                pltpu.make_async_copy(y128_h.at[x2r], g2r, sm).wait()

            def rows1(ck, bu):
                _, _, g1r, g2r, clr, a1r, a2r, _ = bu
                r0c = ck * rc

                def row1(jl):
                    bc = jnp.full((16,), 0, i32) + jl
                    clv = plsc.load_gather(clr, [bc])
                    a1v = plsc.load_gather(a1r, [bc])
                    a2v = plsc.load_gather(a2r, [bc])
                    jbo = (r0c + jl) * cw
                    ylo = jl * cw
                    for v in range(cw // 16):
                        gsv = pl.ds(colbase + v * 16, 16)
                        acc[pl.ds(jbo + v * 16, 16)] = (
                            clv * y_v[pl.ds(ylo + v * 16, 16)]
                            + a1v * g1r[jl, gsv]
                            + a2v * g2r[jl, gsv])

                plsc.parallel_loop(0, rc, unroll=2)(row1)

            prep1(0, bufs[0])
            issue1(bufs[0])

            def pair1(ckp, carry2):
                for p in range(2):
                    ck = ckp * 2 + p
                    nb = bufs[1 - p]

                    @pl.when(ck + 1 < nck)
                    def _():
                        prep1(ck + 1, nb)
                        issue1(nb)

                    pltpu.sync_copy(
                        ysl_h.at[pl.ds((fb + ck * rc) * cw, rc * cw)], y_v)
                    wait1(bufs[p])
                    rows1(ck, bufs[p])
                return carry2

            lax.fori_loop(0, nck // 2, pair1, 0)

            def chunk2(ck, carry2):
                r0c = ck * rc

                def sprep2(sc, carry3):
                    dsl = pl.ds(sc * 16, 16)
                    base = r0c + sc * 16
                    dsg = pl.ds(base, 16)
                    rloc = base + lax.iota(i32, 16)
                    qo = qf_v[dsg]
                    up = jnp.clip(rloc + 1, 0, t - 1)
                    dn = jnp.clip(rloc - 1, 0, t - 1)
                    pu_v[dsl] = jnp.where(rloc < t - 1,
                                          plsc.load_gather(sl_v, [up]),
                                          0.0) * qo
                    pl_v[dsl] = jnp.where(rloc > 0,
                                          plsc.load_gather(su_v, [dn]),
                                          0.0) * qo
                    s1_v[dsl] = u1_v[dsg] * qo
                    s2_v[dsl] = u2_v[dsg] * qo
                    return carry3

                lax.fori_loop(0, rc // 16, sprep2, 0)
                pltpu.sync_copy(ysl_h.at[pl.ds((fb + r0c) * cw, rc * cw)],
                                y_v)

                def row2(sc, carry3):
                    b16 = sc * 16
                    dsl = pl.ds(b16, 16)
                    pu16 = pu_v[dsl]
                    pl16 = pl_v[dsl]
                    s116 = s1_v[dsl]
                    s216 = s2_v[dsl]
                    j116 = i1_v[pl.ds(r0c + b16, 16)]
                    j216 = i2_v[pl.ds(r0c + b16, 16)]
                    for l in range(16):
                        jl = b16 + l
                        jb = r0c + jl
                        ylo = jl * cw
                        ju = jnp.minimum(jb + 1, t - 1)
                        jd = jnp.maximum(jb - 1, 0)
                        for v in range(cw // 16):
                            dsy = pl.ds(ylo + v * 16, 16)
                            yv = y_v[dsy]
                            plsc.addupdate(acc.at[pl.ds(ju * cw + v * 16, 16)],
                                           pu16[l] * yv)
                            plsc.addupdate(acc.at[pl.ds(jd * cw + v * 16, 16)],
                                           pl16[l] * yv)
                        for dst, cf in ((j116, s116), (j216, s216)):
                            cc = cf[l]

                            @pl.when(cc != 0.0)
                            def _():
                                ddo = dst[l] * cw
                                for v in range(cw // 16):
                                    dsa = pl.ds(ddo + v * 16, 16)
                                    dsy = pl.ds(ylo + v * 16, 16)
                                    plsc.addupdate(acc.at[dsa],
                                                   cc * y_v[dsy])
                    return carry3

                lax.fori_loop(0, rc // 16, row2, 0)
                return carry2

            lax.fori_loop(0, t // rc, chunk2, 0)

            def rd(il):
                cf = plsc.load_gather(qf_v, [jnp.full((16,), 0, i32) + il])
                ilo = il * cw
                for v in range(cw // 16):
                    dsv = pl.ds(ilo + v * 16, 16)
                    acc[dsv] = cf * acc[dsv]

            plsc.parallel_loop(0, t, unroll=2)(rd)
            pltpu.sync_copy(acc, out_h.at[pl.ds(fb * cw, t * cw)])
            return carry

        lax.fori_loop(0, tpw, task, 0)

    return k5


def _make_dense_body(cg, cw, sliced_out):
    def body(a_ref, w_ref, b_ref, g_ref, be_ref, *outs):
        h = jnp.concatenate([a_ref[k] for k in range(cg)], axis=1)
        z = lax.dot_general(h, w_ref[...], (((1,), (1,)), ((), ())),
                            preferred_element_type=f32) + b_ref[...]
        z = jnp.where(z > 0, z, jnp.exp(z) - 1.0)
        mu = jnp.mean(z, axis=1, keepdims=True)
        zc = z - mu
        var = jnp.mean(zc * zc, axis=1, keepdims=True)
        o = zc / jnp.sqrt(var + 1e-5) * g_ref[...] + be_ref[...]
        if sliced_out:
            sl_ref, hf_ref = outs
            for k in range(cg):
                sl_ref[k] = o[:, k * cw:(k + 1) * cw]
            hf_ref[0] = o[:, :128]
            hf_ref[1] = o[:, 128:]
        else:
            outs[0][...] = o
    return body


def _dense(agg_sl, w, bb, g, be, sliced_out):
    cg, n, cw = agg_sl.shape
    d = cg * cw
    if sliced_out:
        out_specs = [
            pl.BlockSpec((CG, BR, cw), lambda ib: (0, ib, 0)),
            pl.BlockSpec((2, BR, 128), lambda ib: (0, ib, 0)),
        ]
        out_shape = [
            jax.ShapeDtypeStruct((CG, n, cw), f32),
            jax.ShapeDtypeStruct((2, n, 128), f32),
        ]
    else:
        out_specs = [pl.BlockSpec((BR, d), lambda ib: (ib, 0))]
        out_shape = [jax.ShapeDtypeStruct((n, d), f32)]
    outs = pl.pallas_call(
        _make_dense_body(cg, cw, sliced_out),
        grid=(n // BR,),
        in_specs=[
            pl.BlockSpec((CG, BR, cw), lambda ib: (0, ib, 0)),
            pl.BlockSpec((d, d), lambda ib: (0, 0)),
            pl.BlockSpec((1, d), lambda ib: (0, 0)),
            pl.BlockSpec((1, d), lambda ib: (0, 0)),
            pl.BlockSpec((1, d), lambda ib: (0, 0)),
        ],
        out_specs=out_specs,
        out_shape=out_shape,
    )(agg_sl, w, bb.reshape(1, d), g.reshape(1, d), be.reshape(1, d))
    return outs


def kernel(x, mask, W0, b0, g0, be0, W1, b1, g1, be1):
    b, t, d = x.shape
    bt = b * t
    cw = d // CG
    x2d = x.reshape(bt, d)

    xn, x_sl, x_half = _normalize(x2d)
    xn3 = xn.reshape(b, t, d)
    su = _tridiag(xn3).reshape(b, t)
    sl = jnp.concatenate([jnp.zeros((b, 1), f32), su[:, :-1]], axis=1)
    v1, v2, i1, i2 = _simtop(xn3, b, t, d)

    k3 = _make_k3(b, t)
    rs, w1, w2, u1, u2 = k3(i1, i2, v1, v2, sl, su)

    q = _rsqrt(rs.reshape(b, 1, t)).reshape(b, t)

    k5 = _make_k5(b, t, d)
    i1f = i1.reshape(bt)
    i2f = i2.reshape(bt)
    slf = sl.reshape(bt)
    suf = su.reshape(bt)
    w1f = w1.reshape(bt)
    w2f = w2.reshape(bt)
    u1f = u1.reshape(bt)
    u2f = u2.reshape(bt)

    def _agg(y_sl, y_half):
        return k5(y_sl.reshape(CG * bt * cw), y_half.reshape(2 * bt, 128),
                  q, i1f, i2f, slf, suf, w1f, w2f, u1f, u2f)

    agg1 = _agg(x_sl, x_half).reshape(CG, bt, cw)
    h1_sl, h1_half = _dense(agg1, W0, b0, g0, be0, sliced_out=True)
    agg2 = _agg(h1_sl, h1_half).reshape(CG, bt, cw)
    (h2,) = _dense(agg2, W1, b1, g1, be1, sliced_out=False)
    return h2.reshape(b, t, d)
